# Initial kernel scaffold; baseline (speedup 1.0000x reference)
import jax
import jax.numpy as jnp
from jax.experimental import pallas as pl


def kernel(feat_invar, feat_var, last_nodes, Wu, bu, Wv, We):
    raise NotImplementedError("write your pallas kernel here")


# speedup vs baseline: 40.9774x; 40.9774x over previous
import functools

import jax
import jax.numpy as jnp
from jax import lax
from jax.experimental import pallas as pl
from jax.experimental.pallas import tpu as pltpu
from jax.experimental.pallas import tpu_sc as plsc

B = 1000
NPG = 100
N = B * NPG
D = 128
H = 128

GB = 40
R = GB * NPG
GRID = B // GB

_NC, _NS = 2, 16
_NW = _NC * _NS
BP = 1024
BPW = BP // _NW


@functools.cache
def _get_sc_gather():
    mesh = plsc.VectorSubcoreMesh(core_axis_name="c", subcore_axis_name="s")

    @functools.partial(
        pl.kernel,
        mesh=mesh,
        out_type=[
            jax.ShapeDtypeStruct((BP, D), jnp.float32),
            jax.ShapeDtypeStruct((BP, D), jnp.float32),
        ],
        scratch_types=[
            pltpu.VMEM((BPW,), jnp.int32),
            pltpu.VMEM((BPW, D), jnp.float32),
            pltpu.VMEM((BPW, D), jnp.float32),
            pltpu.SemaphoreType.DMA,
            pltpu.SemaphoreType.DMA,
        ],
    )
    def _sc_gather(ti_hbm, tv_hbm, idx_hbm, oi_hbm, ov_hbm,
                   idx_v, ri_v, rv_v, s1, s2):
        wid = lax.axis_index("s") * _NC + lax.axis_index("c")
        base = wid * BPW
        pltpu.sync_copy(idx_hbm.at[pl.ds(base, BPW)], idx_v)
        c1 = pltpu.async_copy(ti_hbm.at[idx_v], ri_v, s1)
        c2 = pltpu.async_copy(tv_hbm.at[idx_v], rv_v, s2)
        c1.wait()
        c2.wait()
        pltpu.sync_copy(ri_v, oi_hbm.at[pl.ds(base, BPW)])
        pltpu.sync_copy(rv_v, ov_hbm.at[pl.ds(base, BPW)])

    return _sc_gather


def _attn_block(xi_ref, xv_ref, fvi_ref, fvv_ref, wu_ref, bu_ref, wv_ref,
                we_ref, oi_ref, ov_ref):
    xi = xi_ref[...]
    xv = xv_ref[...]
    wu = wu_ref[...]
    bu = bu_ref[...]
    ui = jnp.dot(xi, wu, preferred_element_type=jnp.float32) + bu
    uv = jnp.dot(xv, wu, preferred_element_type=jnp.float32) + bu
    qi = jnp.dot(fvi_ref[...], wv_ref[...], preferred_element_type=jnp.float32)
    qv = jnp.dot(fvv_ref[...], wv_ref[...], preferred_element_type=jnp.float32)

    row_g = lax.broadcasted_iota(jnp.int32, (R, GB), 0) // NPG
    col_g = lax.broadcasted_iota(jnp.int32, (R, GB), 1)
    smat = (row_g == col_g).astype(jnp.float32)
    g_row = lax.broadcasted_iota(jnp.int32, (GB, R), 0)
    row2_g = lax.broadcasted_iota(jnp.int32, (GB, R), 1) // NPG
    smat_t = (g_row == row2_g).astype(jnp.float32)

    we_row = we_ref[...]

    def one_query(q, out_ref):
        qrows = jnp.dot(smat, q, preferred_element_type=jnp.float32)
        ei = jnp.sum(jax.nn.sigmoid(ui + qrows) * we_row, axis=1,
                     keepdims=True)
        ev = jnp.sum(jax.nn.sigmoid(uv + qrows) * we_row, axis=1,
                     keepdims=True)
        m = jnp.max(jnp.maximum(ei, ev))
        wi = jnp.exp(ei - m)
        wv_ = jnp.exp(ev - m)
        denom = jnp.dot(smat_t, wi + wv_,
                        preferred_element_type=jnp.float32)
        num = jnp.dot(smat_t, xi * wi + xv * wv_,
                      preferred_element_type=jnp.float32)
        out_ref[...] = num / denom

    one_query(qi, oi_ref)
    one_query(qv, ov_ref)


def _tc_call(feat_invar, feat_var, fvi, fvv, wu, bu, wv, we):
    full = lambda shape: pl.BlockSpec(shape, lambda i: (0, 0))
    return pl.pallas_call(
        _attn_block,
        grid=(GRID,),
        in_specs=[
            pl.BlockSpec((R, D), lambda i: (i, 0)),
            pl.BlockSpec((R, D), lambda i: (i, 0)),
            pl.BlockSpec((GB, D), lambda i: (i, 0)),
            pl.BlockSpec((GB, D), lambda i: (i, 0)),
            full((D, H)),
            full((1, H)),
            full((D, H)),
            full((1, H)),
        ],
        out_specs=[
            pl.BlockSpec((GB, D), lambda i: (i, 0)),
            pl.BlockSpec((GB, D), lambda i: (i, 0)),
        ],
        out_shape=[
            jax.ShapeDtypeStruct((B, D), jnp.float32),
            jax.ShapeDtypeStruct((B, D), jnp.float32),
        ],
    )(feat_invar, feat_var, fvi, fvv, wu, bu, wv, we)


def kernel(feat_invar, feat_var, last_nodes, Wu, bu, Wv, We):
    idx = jnp.pad(last_nodes.astype(jnp.int32), (0, BP - B))
    fvi, fvv = _get_sc_gather()(feat_invar, feat_var, idx)
    rst_i, rst_v = _tc_call(feat_invar, feat_var, fvi, fvv,
                            Wu, bu.reshape(1, H), Wv, We.reshape(1, H))
    return rst_i[:, None, :], rst_v[:, None, :]
